# Initial kernel scaffold; baseline (speedup 1.0000x reference)
import jax
import jax.numpy as jnp
from jax.experimental import pallas as pl


def kernel(loc_data, conf_data, priors, targets):
    raise NotImplementedError("write your pallas kernel here")


# speedup vs baseline: 74.6278x; 74.6278x over previous
import jax
import jax.numpy as jnp
from jax.experimental import pallas as pl
from jax.experimental.pallas import tpu as pltpu

_NUM_CLASSES = 21
_THRESHOLD = 0.5
_NEGPOS_RATIO = 3
_VAR0, _VAR1 = 0.1, 0.2
_B, _P, _NOBJ = 32, 24532, 32
_PPAD = 24576
_PB = 2048
_NCH = _PPAD // _PB


def _stage1_body(tgt_ref, pr_ref, locT_ref, confT_ref,
                 mined_ref, ll_ref, npos_ref, spos_ref,
                 bto_s, bti_s):
    f32 = jnp.float32
    t = tgt_ref[0]
    tx1, ty1 = t[:, 0:1], t[:, 1:2]
    tx2, ty2 = t[:, 2:3], t[:, 3:4]
    lab = t[:, 4:5]
    area_t = (tx2 - tx1) * (ty2 - ty1)
    jcol = jax.lax.broadcasted_iota(jnp.int32, (_NOBJ, 1), 0).astype(f32)

    runmax = jnp.full((_NOBJ, 1), -jnp.inf, f32)
    runidx = jnp.zeros((_NOBJ, 1), f32)

    for c in range(_NCH):
        sl = pl.ds(c * _PB, _PB)
        cx, cy = pr_ref[0:1, sl], pr_ref[1:2, sl]
        w, h = pr_ref[2:3, sl], pr_ref[3:4, sl]
        px1, py1 = cx - w * 0.5, cy - h * 0.5
        px2, py2 = w + w * 0.5, h + h * 0.5
        area_p = (px2 - px1) * (py2 - py1)
        ix = jnp.maximum(jnp.minimum(tx2, px2) - jnp.maximum(tx1, px1), 0.0)
        iy = jnp.maximum(jnp.minimum(ty2, py2) - jnp.maximum(ty1, py1), 0.0)
        inter = ix * iy
        ov = inter / (area_t + area_p - inter)
        bto_c = jnp.max(ov, axis=0, keepdims=True)
        bti_c = jnp.min(jnp.where(ov == bto_c, jcol, 99.0), axis=0, keepdims=True)
        bto_s[:, sl] = bto_c
        bti_s[:, sl] = bti_c
        gidx = jax.lax.broadcasted_iota(jnp.int32, (1, _PB), 1).astype(f32) + float(c * _PB)
        rm_c = jnp.max(ov, axis=1, keepdims=True)
        ri_c = jnp.min(jnp.where(ov == rm_c, gidx, 1e9), axis=1, keepdims=True)
        upd = rm_c > runmax
        runmax = jnp.where(upd, rm_c, runmax)
        runidx = jnp.where(upd, ri_c, runidx)

    bpi = runidx

    ll = jnp.zeros((), f32)
    np_ = jnp.zeros((), f32)
    sp = jnp.zeros((), f32)
    clsio = jax.lax.broadcasted_iota(jnp.int32, (_NUM_CLASSES, 1), 0).astype(f32)

    for c in range(_NCH):
        sl = pl.ds(c * _PB, _PB)
        gidx = jax.lax.broadcasted_iota(jnp.int32, (1, _PB), 1).astype(f32) + float(c * _PB)
        bto = bto_s[:, sl]
        bti = bti_s[:, sl]
        hit = gidx == bpi
        forcej = jnp.max(jnp.where(hit, jcol, -1.0), axis=0, keepdims=True)
        forced = forcej >= 0.0
        bto = jnp.where(forced, 2.0, bto)
        bti = jnp.where(forced, forcej, bti)
        selm = bti == jcol
        labm = jnp.sum(jnp.where(selm, lab, 0.0), axis=0, keepdims=True)
        conf = jnp.where(bto < _THRESHOLD, 0.0, labm + 1.0)
        pos = conf > 0.0
        posf = pos.astype(f32)
        mx1 = jnp.sum(jnp.where(selm, tx1, 0.0), axis=0, keepdims=True)
        my1 = jnp.sum(jnp.where(selm, ty1, 0.0), axis=0, keepdims=True)
        mx2 = jnp.sum(jnp.where(selm, tx2, 0.0), axis=0, keepdims=True)
        my2 = jnp.sum(jnp.where(selm, ty2, 0.0), axis=0, keepdims=True)
        cx, cy = pr_ref[0:1, sl], pr_ref[1:2, sl]
        w, h = pr_ref[2:3, sl], pr_ref[3:4, sl]
        g0 = (mx1 + mx2 * 0.5 - cx) / (_VAR0 * w)
        g1 = (my1 + my2 * 0.5 - cy) / (_VAR0 * h)
        g2 = jnp.log((mx2 - mx1) / w) / _VAR1
        g3 = jnp.log((my2 - my1) / h) / _VAR1
        loct = jnp.concatenate([g0, g1, g2, g3], axis=0)
        locd = locT_ref[0, :, sl]
        ad = jnp.abs(locd - loct)
        sl1 = jnp.where(ad < 1.0, 0.5 * ad * ad, ad - 0.5)
        ll = ll + jnp.sum(sl1 * posf)
        np_ = np_ + jnp.sum(posf)
        x = confT_ref[0, :, sl]
        m = jnp.max(x, axis=0, keepdims=True)
        s = jnp.sum(jnp.exp(x - m), axis=0, keepdims=True)
        lse = jnp.log(s) + m
        tgt = jnp.sum(jnp.where(clsio == conf, x, 0.0), axis=0, keepdims=True)
        ce = lse - tgt
        mined = jnp.where(pos, 0.0, ce)
        mined = jnp.where(gidx < float(_P), mined, 0.0)
        mined_ref[0, :, sl] = mined
        sp = sp + jnp.sum(jnp.where(pos, ce, 0.0))

    ll_ref[...] = ll.reshape(1, 1, 1)
    npos_ref[...] = np_.reshape(1, 1, 1)
    spos_ref[...] = sp.reshape(1, 1, 1)


def _stage2_body(mined_ref, ll_ref, npos_ref, spos_ref, outl_ref, outc_ref):
    f32 = jnp.float32
    mined = mined_ref[...]
    npos = npos_ref[...]
    k = jnp.minimum(float(_NEGPOS_RATIO) * npos, float(_P - 1))
    bits = jax.lax.bitcast_convert_type(mined, jnp.int32)
    u = jnp.zeros((_B, 1, 1), jnp.int32)
    for b in range(30, -1, -1):
        trial = u | (1 << b)
        cnt = jnp.sum((bits >= trial).astype(f32), axis=2, keepdims=True)
        u = jnp.where(cnt >= k, trial, u)
    gt = bits > u
    cgt = jnp.sum(gt.astype(f32), axis=2, keepdims=True)
    sgt = jnp.sum(jnp.where(gt, mined, 0.0), axis=2, keepdims=True)
    tau = jax.lax.bitcast_convert_type(u, f32)
    sneg = sgt + (k - cgt) * tau
    n = jnp.sum(npos)
    outl_ref[...] = (jnp.sum(ll_ref[...]) / n).reshape(1, 1)
    outc_ref[...] = ((jnp.sum(spos_ref[...]) + jnp.sum(sneg)) / n).reshape(1, 1)


def kernel(loc_data, conf_data, priors, targets):
    pad = _PPAD - _P
    pad_pr = jnp.tile(jnp.array([[5.0, 5.0, 0.001, 0.001]], jnp.float32), (pad, 1))
    priorsT = jnp.concatenate([priors, pad_pr], axis=0).T
    locT = jnp.pad(loc_data, ((0, 0), (0, pad), (0, 0))).transpose(0, 2, 1)
    confT = jnp.pad(conf_data, ((0, 0), (0, pad), (0, 0))).transpose(0, 2, 1)

    mined, ll, npos, spos = pl.pallas_call(
        _stage1_body,
        grid=(_B,),
        in_specs=[
            pl.BlockSpec((1, _NOBJ, 5), lambda b: (b, 0, 0)),
            pl.BlockSpec((4, _PPAD), lambda b: (0, 0)),
            pl.BlockSpec((1, 4, _PPAD), lambda b: (b, 0, 0)),
            pl.BlockSpec((1, _NUM_CLASSES, _PPAD), lambda b: (b, 0, 0)),
        ],
        out_specs=[
            pl.BlockSpec((1, 1, _PPAD), lambda b: (b, 0, 0)),
            pl.BlockSpec((1, 1, 1), lambda b: (b, 0, 0)),
            pl.BlockSpec((1, 1, 1), lambda b: (b, 0, 0)),
            pl.BlockSpec((1, 1, 1), lambda b: (b, 0, 0)),
        ],
        out_shape=[
            jax.ShapeDtypeStruct((_B, 1, _PPAD), jnp.float32),
            jax.ShapeDtypeStruct((_B, 1, 1), jnp.float32),
            jax.ShapeDtypeStruct((_B, 1, 1), jnp.float32),
            jax.ShapeDtypeStruct((_B, 1, 1), jnp.float32),
        ],
        scratch_shapes=[
            pltpu.VMEM((1, _PPAD), jnp.float32),
            pltpu.VMEM((1, _PPAD), jnp.float32),
        ],
    )(targets, priorsT, locT, confT)

    outl, outc = pl.pallas_call(
        _stage2_body,
        out_shape=[
            jax.ShapeDtypeStruct((1, 1), jnp.float32),
            jax.ShapeDtypeStruct((1, 1), jnp.float32),
        ],
    )(mined, ll, npos, spos)
    return outl[0, 0], outc[0, 0]


# speedup vs baseline: 83.5505x; 1.1196x over previous
import jax
import jax.numpy as jnp
from jax import lax
from jax.experimental import pallas as pl
from jax.experimental.pallas import tpu as pltpu
from jax.experimental.pallas import tpu_sc as plsc

_NUM_CLASSES = 21
_THRESHOLD = 0.5
_NEGPOS_RATIO = 3
_VAR0, _VAR1 = 0.1, 0.2
_B, _P, _NOBJ = 32, 24532, 32
_PPAD = 24576
_PB = 2048
_NCH = _PPAD // _PB

_i32 = jnp.int32
_f32 = jnp.float32


def _stage1_body(tgt_ref, pr_ref, locT_ref, confT_ref,
                 mined_ref, ll_ref, npos_ref, spos_ref,
                 bto_s, bti_s):
    f32 = jnp.float32
    t = tgt_ref[0]
    tx1, ty1 = t[:, 0:1], t[:, 1:2]
    tx2, ty2 = t[:, 2:3], t[:, 3:4]
    lab = t[:, 4:5]
    area_t = (tx2 - tx1) * (ty2 - ty1)
    jcol = jax.lax.broadcasted_iota(jnp.int32, (_NOBJ, 1), 0).astype(f32)

    runmax = jnp.full((_NOBJ, 1), -jnp.inf, f32)
    runidx = jnp.zeros((_NOBJ, 1), f32)

    for c in range(_NCH):
        sl = pl.ds(c * _PB, _PB)
        cx, cy = pr_ref[0:1, sl], pr_ref[1:2, sl]
        w, h = pr_ref[2:3, sl], pr_ref[3:4, sl]
        px1, py1 = cx - w * 0.5, cy - h * 0.5
        px2, py2 = w + w * 0.5, h + h * 0.5
        area_p = (px2 - px1) * (py2 - py1)
        ix = jnp.maximum(jnp.minimum(tx2, px2) - jnp.maximum(tx1, px1), 0.0)
        iy = jnp.maximum(jnp.minimum(ty2, py2) - jnp.maximum(ty1, py1), 0.0)
        inter = ix * iy
        ov = inter / (area_t + area_p - inter)
        bto_c = jnp.max(ov, axis=0, keepdims=True)
        bti_c = jnp.min(jnp.where(ov == bto_c, jcol, 99.0), axis=0, keepdims=True)
        bto_s[:, sl] = bto_c
        bti_s[:, sl] = bti_c
        gidx = jax.lax.broadcasted_iota(jnp.int32, (1, _PB), 1).astype(f32) + float(c * _PB)
        rm_c = jnp.max(ov, axis=1, keepdims=True)
        ri_c = jnp.min(jnp.where(ov == rm_c, gidx, 1e9), axis=1, keepdims=True)
        upd = rm_c > runmax
        runmax = jnp.where(upd, rm_c, runmax)
        runidx = jnp.where(upd, ri_c, runidx)

    bpi = runidx

    ll = jnp.zeros((), f32)
    np_ = jnp.zeros((), f32)
    sp = jnp.zeros((), f32)
    clsio = jax.lax.broadcasted_iota(jnp.int32, (_NUM_CLASSES, 1), 0).astype(f32)

    for c in range(_NCH):
        sl = pl.ds(c * _PB, _PB)
        gidx = jax.lax.broadcasted_iota(jnp.int32, (1, _PB), 1).astype(f32) + float(c * _PB)
        bto = bto_s[:, sl]
        bti = bti_s[:, sl]
        hit = gidx == bpi
        forcej = jnp.max(jnp.where(hit, jcol, -1.0), axis=0, keepdims=True)
        forced = forcej >= 0.0
        bto = jnp.where(forced, 2.0, bto)
        bti = jnp.where(forced, forcej, bti)
        selm = bti == jcol
        labm = jnp.sum(jnp.where(selm, lab, 0.0), axis=0, keepdims=True)
        conf = jnp.where(bto < _THRESHOLD, 0.0, labm + 1.0)
        pos = conf > 0.0
        posf = pos.astype(f32)
        mx1 = jnp.sum(jnp.where(selm, tx1, 0.0), axis=0, keepdims=True)
        my1 = jnp.sum(jnp.where(selm, ty1, 0.0), axis=0, keepdims=True)
        mx2 = jnp.sum(jnp.where(selm, tx2, 0.0), axis=0, keepdims=True)
        my2 = jnp.sum(jnp.where(selm, ty2, 0.0), axis=0, keepdims=True)
        cx, cy = pr_ref[0:1, sl], pr_ref[1:2, sl]
        w, h = pr_ref[2:3, sl], pr_ref[3:4, sl]
        g0 = (mx1 + mx2 * 0.5 - cx) / (_VAR0 * w)
        g1 = (my1 + my2 * 0.5 - cy) / (_VAR0 * h)
        g2 = jnp.log((mx2 - mx1) / w) / _VAR1
        g3 = jnp.log((my2 - my1) / h) / _VAR1
        loct = jnp.concatenate([g0, g1, g2, g3], axis=0)
        locd = locT_ref[0, :, sl]
        ad = jnp.abs(locd - loct)
        sl1 = jnp.where(ad < 1.0, 0.5 * ad * ad, ad - 0.5)
        ll = ll + jnp.sum(sl1 * posf)
        np_ = np_ + jnp.sum(posf)
        x = confT_ref[0, :, sl]
        m = jnp.max(x, axis=0, keepdims=True)
        s = jnp.sum(jnp.exp(x - m), axis=0, keepdims=True)
        lse = jnp.log(s) + m
        tgt = jnp.sum(jnp.where(clsio == conf, x, 0.0), axis=0, keepdims=True)
        ce = lse - tgt
        mined = jnp.where(pos, 0.0, ce)
        mined = jnp.where(gidx < float(_P), mined, 0.0)
        mined_ref[0, :, sl] = mined
        sp = sp + jnp.sum(jnp.where(pos, ce, 0.0))

    ll_ref[...] = ll.reshape(1, 1, 1)
    npos_ref[...] = np_.reshape(1, 1, 1)
    spos_ref[...] = sp.reshape(1, 1, 1)


_NV = _PPAD // 16
_BASE1, _BASE2, _BASE3 = 0, 2048, 4096
_HTOT = 4608


def _scsum(v):
    return lax.reduce_sum(v, (0,))


def _lane_extract(vec, lane):
    io = lax.broadcasted_iota(_i32, (16,), 0)
    return _scsum(jnp.where(io == lane, vec, jnp.zeros_like(vec)))


def _locate(hist_v, base, nbuckets, rank):
    nv = nbuckets // 16

    def body(i, carry):
        found, beta, rankrem, acc = carry
        vi = nv - 1 - i
        h = hist_v[pl.ds(base + vi * 16, 16)]
        s = lax.rev(lax.cumsum(lax.rev(h, (0,)), axis=0), (0,))
        suff = s + acc
        m = suff >= rank
        t = _scsum(jnp.where(m, jnp.ones((16,), _i32), jnp.zeros((16,), _i32)))
        hit = jnp.logical_and(found == 0, t > 0)
        lane = t - 1
        sl = _lane_extract(s, lane)
        hl = _lane_extract(h, lane)
        beta_new = vi * 16 + lane
        rr_new = rank - (acc + sl - hl)
        found = jnp.where(hit, 1, found)
        beta = jnp.where(hit, beta_new, beta)
        rankrem = jnp.where(hit, rr_new, rankrem)
        acc = acc + _scsum(h)
        return found, beta, rankrem, acc

    z = jnp.zeros((), _i32)
    found, beta, rankrem, acc = lax.fori_loop(0, nv, body, (z, z, z, z))
    return beta, rankrem


def _sc_topk_body(mined_hbm, npos_hbm, out_hbm, mined_v, npos_v, hist_v, outbuf_v):
    wid = lax.axis_index("s") * 2 + lax.axis_index("c")
    pltpu.sync_copy(mined_hbm.at[wid], mined_v)
    pltpu.sync_copy(npos_hbm, npos_v)

    lane = jnp.mod(wid, 16)
    half = wid // 16
    va = npos_v[pl.ds(0, 16)]
    vb = npos_v[pl.ds(16, 16)]
    hsel = jnp.full((16,), half, _i32) == 0
    nv16 = jnp.where(hsel, va, vb)
    npf = _scsum(jnp.where(lax.broadcasted_iota(_i32, (16,), 0) == lane,
                           nv16, jnp.zeros((16,), _f32)))
    k = jnp.minimum(_NEGPOS_RATIO * npf.astype(_i32), _P - 1)

    def zbody(i, c):
        hist_v[pl.ds(i * 16, 16)] = jnp.zeros((16,), _i32)
        return c

    lax.fori_loop(0, _HTOT // 16, zbody, 0)

    ones = jnp.ones((16,), _i32)

    def p1(i, c):
        v = mined_v[pl.ds(i * 16, 16)]
        b = lax.shift_right_logical(plsc.bitcast(v, _i32), 20)
        plsc.addupdate_scatter(hist_v, [b + _BASE1], ones)
        return c

    lax.fori_loop(0, _NV, p1, 0)
    b1, r2 = _locate(hist_v, _BASE1, 2048, k)

    def p2(i, c):
        v = mined_v[pl.ds(i * 16, 16)]
        bits = plsc.bitcast(v, _i32)
        pre = lax.shift_right_logical(bits, 20)
        b = jnp.bitwise_and(lax.shift_right_logical(bits, 9), 0x7FF)
        plsc.addupdate_scatter(hist_v, [b + _BASE2], ones, mask=pre == b1)
        return c

    lax.fori_loop(0, _NV, p2, 0)
    b2, r3 = _locate(hist_v, _BASE2, 2048, r2)

    pre22 = jnp.bitwise_or(lax.shift_left(b1, 11), b2)

    def p3(i, c):
        v = mined_v[pl.ds(i * 16, 16)]
        bits = plsc.bitcast(v, _i32)
        pre = lax.shift_right_logical(bits, 9)
        b = jnp.bitwise_and(bits, 0x1FF)
        plsc.addupdate_scatter(hist_v, [b + _BASE3], ones, mask=pre == pre22)
        return c

    lax.fori_loop(0, _NV, p3, 0)
    b3, _r4 = _locate(hist_v, _BASE3, 512, r3)

    tau_bits = jnp.bitwise_or(lax.shift_left(b1, 20),
                              jnp.bitwise_or(lax.shift_left(b2, 9), b3))

    def p4(i, carry):
        cs, ss = carry
        v = mined_v[pl.ds(i * 16, 16)]
        bits = plsc.bitcast(v, _i32)
        m = bits > tau_bits
        cs = cs + jnp.where(m, ones, jnp.zeros((16,), _i32))
        ss = ss + jnp.where(m, v, jnp.zeros((16,), _f32))
        return cs, ss

    cs, ss = lax.fori_loop(0, _NV, p4,
                           (jnp.zeros((16,), _i32), jnp.zeros((16,), _f32)))
    cgt = _scsum(cs)
    sgt = _scsum(ss)
    tau_v = plsc.bitcast(jnp.full((16,), tau_bits, _i32), _f32)
    res = jnp.full((16,), sgt, _f32) + \
        jnp.full((16,), (k - cgt).astype(_f32), _f32) * tau_v
    outbuf_v[...] = res
    pltpu.sync_copy(outbuf_v, out_hbm.at[wid])


def _sc_topk(mined, npos):
    mesh = plsc.VectorSubcoreMesh(core_axis_name="c", subcore_axis_name="s",
                                  num_cores=2, num_subcores=16)
    return pl.kernel(
        _sc_topk_body,
        out_type=jax.ShapeDtypeStruct((_B, 16), _f32),
        mesh=mesh,
        compiler_params=pltpu.CompilerParams(needs_layout_passes=False),
        scratch_types=[
            pltpu.VMEM((_PPAD,), _f32),
            pltpu.VMEM((_B,), _f32),
            pltpu.VMEM((_HTOT,), _i32),
            pltpu.VMEM((16,), _f32),
        ],
    )(mined, npos)


def _stage3_body(ll_ref, npos_ref, spos_ref, sneg_ref, outl_ref, outc_ref):
    n = jnp.sum(npos_ref[...])
    sneg = sneg_ref[...][:, 0:1]
    outl_ref[...] = (jnp.sum(ll_ref[...]) / n).reshape(1, 1)
    outc_ref[...] = ((jnp.sum(spos_ref[...]) + jnp.sum(sneg)) / n).reshape(1, 1)


def kernel(loc_data, conf_data, priors, targets):
    pad = _PPAD - _P
    pad_pr = jnp.tile(jnp.array([[5.0, 5.0, 0.001, 0.001]], jnp.float32), (pad, 1))
    priorsT = jnp.concatenate([priors, pad_pr], axis=0).T
    locT = jnp.pad(loc_data, ((0, 0), (0, pad), (0, 0))).transpose(0, 2, 1)
    confT = jnp.pad(conf_data, ((0, 0), (0, pad), (0, 0))).transpose(0, 2, 1)

    mined, ll, npos, spos = pl.pallas_call(
        _stage1_body,
        grid=(_B,),
        in_specs=[
            pl.BlockSpec((1, _NOBJ, 5), lambda b: (b, 0, 0)),
            pl.BlockSpec((4, _PPAD), lambda b: (0, 0)),
            pl.BlockSpec((1, 4, _PPAD), lambda b: (b, 0, 0)),
            pl.BlockSpec((1, _NUM_CLASSES, _PPAD), lambda b: (b, 0, 0)),
        ],
        out_specs=[
            pl.BlockSpec((1, 1, _PPAD), lambda b: (b, 0, 0)),
            pl.BlockSpec((1, 1, 1), lambda b: (b, 0, 0)),
            pl.BlockSpec((1, 1, 1), lambda b: (b, 0, 0)),
            pl.BlockSpec((1, 1, 1), lambda b: (b, 0, 0)),
        ],
        out_shape=[
            jax.ShapeDtypeStruct((_B, 1, _PPAD), jnp.float32),
            jax.ShapeDtypeStruct((_B, 1, 1), jnp.float32),
            jax.ShapeDtypeStruct((_B, 1, 1), jnp.float32),
            jax.ShapeDtypeStruct((_B, 1, 1), jnp.float32),
        ],
        scratch_shapes=[
            pltpu.VMEM((1, _PPAD), jnp.float32),
            pltpu.VMEM((1, _PPAD), jnp.float32),
        ],
    )(targets, priorsT, locT, confT)

    sneg = _sc_topk(mined.reshape(_B, _PPAD), npos.reshape(_B))

    outl, outc = pl.pallas_call(
        _stage3_body,
        out_shape=[
            jax.ShapeDtypeStruct((1, 1), jnp.float32),
            jax.ShapeDtypeStruct((1, 1), jnp.float32),
        ],
    )(ll, npos, spos, sneg)
    return outl[0, 0], outc[0, 0]


# speedup vs baseline: 100.5092x; 1.2030x over previous
import jax
import jax.numpy as jnp
from jax import lax
from jax.experimental import pallas as pl
from jax.experimental.pallas import tpu as pltpu
from jax.experimental.pallas import tpu_sc as plsc

_NUM_CLASSES = 21
_THRESHOLD = 0.5
_NEGPOS_RATIO = 3
_VAR0, _VAR1 = 0.1, 0.2
_B, _P, _NOBJ = 32, 24532, 32
_PPAD = 24576
_PB = 2048
_NCH = _PPAD // _PB

_i32 = jnp.int32
_f32 = jnp.float32


def _stage1_body(tgt_ref, tgtT_ref, pr_ref, locT_ref, confT_ref,
                 mined_ref, ll_ref, npos_ref, spos_ref,
                 bto_s, bti_s):
    f32 = jnp.float32
    t = tgt_ref[0]
    t5 = tgtT_ref[0]
    tx1, ty1 = t[:, 0:1], t[:, 1:2]
    tx2, ty2 = t[:, 2:3], t[:, 3:4]
    lab = t[:, 4:5]
    area_t = (tx2 - tx1) * (ty2 - ty1)
    jcol = jax.lax.broadcasted_iota(jnp.int32, (_NOBJ, 1), 0).astype(f32)

    runmax = jnp.full((_NOBJ, 1), -jnp.inf, f32)
    runidx = jnp.zeros((_NOBJ, 1), f32)

    for c in range(_NCH):
        sl = pl.ds(c * _PB, _PB)
        cx, cy = pr_ref[0:1, sl], pr_ref[1:2, sl]
        w, h = pr_ref[2:3, sl], pr_ref[3:4, sl]
        px1, py1 = cx - w * 0.5, cy - h * 0.5
        px2, py2 = w + w * 0.5, h + h * 0.5
        area_p = (px2 - px1) * (py2 - py1)
        ix = jnp.maximum(jnp.minimum(tx2, px2) - jnp.maximum(tx1, px1), 0.0)
        iy = jnp.maximum(jnp.minimum(ty2, py2) - jnp.maximum(ty1, py1), 0.0)
        inter = ix * iy
        ov = inter / (area_t + area_p - inter)
        bto_c = jnp.max(ov, axis=0, keepdims=True)
        bti_c = jnp.min(jnp.where(ov == bto_c, jcol, 99.0), axis=0, keepdims=True)
        bto_s[:, sl] = bto_c
        bti_s[:, sl] = bti_c
        gidx = jax.lax.broadcasted_iota(jnp.int32, (1, _PB), 1).astype(f32) + float(c * _PB)
        rm_c = jnp.max(ov, axis=1, keepdims=True)
        ri_c = jnp.min(jnp.where(ov == rm_c, gidx, 1e9), axis=1, keepdims=True)
        upd = rm_c > runmax
        runmax = jnp.where(upd, rm_c, runmax)
        runidx = jnp.where(upd, ri_c, runidx)

    bpi = runidx

    ll = jnp.zeros((), f32)
    np_ = jnp.zeros((), f32)
    sp = jnp.zeros((), f32)
    clsio = jax.lax.broadcasted_iota(jnp.int32, (_NUM_CLASSES, 1), 0).astype(f32)

    for c in range(_NCH):
        sl = pl.ds(c * _PB, _PB)
        gidx = jax.lax.broadcasted_iota(jnp.int32, (1, _PB), 1).astype(f32) + float(c * _PB)
        bto = bto_s[:, sl]
        bti = bti_s[:, sl]
        hit = gidx == bpi
        forcej = jnp.max(jnp.where(hit, jcol, -1.0), axis=0, keepdims=True)
        forced = forcej >= 0.0
        bto = jnp.where(forced, 2.0, bto)
        bti = jnp.where(forced, forcej, bti)
        selm = (bti == jcol).astype(f32)
        coords = jax.lax.dot_general(t5, selm, (((1,), (0,)), ((), ())),
                                     preferred_element_type=f32)
        mx1, my1 = coords[0:1, :], coords[1:2, :]
        mx2, my2 = coords[2:3, :], coords[3:4, :]
        labm = coords[4:5, :]
        conf = jnp.where(bto < _THRESHOLD, 0.0, labm + 1.0)
        pos = conf > 0.0
        posf = pos.astype(f32)
        cx, cy = pr_ref[0:1, sl], pr_ref[1:2, sl]
        w, h = pr_ref[2:3, sl], pr_ref[3:4, sl]
        g0 = (mx1 + mx2 * 0.5 - cx) / (_VAR0 * w)
        g1 = (my1 + my2 * 0.5 - cy) / (_VAR0 * h)
        g2 = jnp.log((mx2 - mx1) / w) / _VAR1
        g3 = jnp.log((my2 - my1) / h) / _VAR1
        loct = jnp.concatenate([g0, g1, g2, g3], axis=0)
        locd = locT_ref[0, :, sl]
        ad = jnp.abs(locd - loct)
        sl1 = jnp.where(ad < 1.0, 0.5 * ad * ad, ad - 0.5)
        ll = ll + jnp.sum(sl1 * posf)
        np_ = np_ + jnp.sum(posf)
        x = confT_ref[0, :, sl]
        m = jnp.max(x, axis=0, keepdims=True)
        s = jnp.sum(jnp.exp(x - m), axis=0, keepdims=True)
        lse = jnp.log(s) + m
        tgt = jnp.sum(jnp.where(clsio == conf, x, 0.0), axis=0, keepdims=True)
        ce = lse - tgt
        mined = jnp.where(pos, 0.0, ce)
        mined = jnp.where(gidx < float(_P), mined, 0.0)
        mined_ref[0, :, sl] = mined
        sp = sp + jnp.sum(jnp.where(pos, ce, 0.0))

    ll_ref[...] = ll.reshape(1, 1, 1)
    npos_ref[...] = np_.reshape(1, 1, 1)
    spos_ref[...] = sp.reshape(1, 1, 1)


_NV = _PPAD // 16
_BASE1, _BASE2, _BASE3 = 0, 2048, 4096
_HTOT = 4608


def _scsum(v):
    return lax.reduce_sum(v, (0,))


def _lane_extract(vec, lane):
    io = lax.broadcasted_iota(_i32, (16,), 0)
    return _scsum(jnp.where(io == lane, vec, jnp.zeros_like(vec)))


def _locate(hist_v, base, nbuckets, rank):
    nv = nbuckets // 16

    def body(i, carry):
        found, beta, rankrem, acc = carry
        vi = nv - 1 - i
        h = hist_v[pl.ds(base + vi * 16, 16)]
        s = lax.rev(lax.cumsum(lax.rev(h, (0,)), axis=0), (0,))
        suff = s + acc
        m = suff >= rank
        t = _scsum(jnp.where(m, jnp.ones((16,), _i32), jnp.zeros((16,), _i32)))
        hit = jnp.logical_and(found == 0, t > 0)
        lane = t - 1
        sl = _lane_extract(s, lane)
        hl = _lane_extract(h, lane)
        beta_new = vi * 16 + lane
        rr_new = rank - (acc + sl - hl)
        found = jnp.where(hit, 1, found)
        beta = jnp.where(hit, beta_new, beta)
        rankrem = jnp.where(hit, rr_new, rankrem)
        acc = acc + _scsum(h)
        return found, beta, rankrem, acc

    z = jnp.zeros((), _i32)
    found, beta, rankrem, acc = lax.fori_loop(0, nv, body, (z, z, z, z))
    return beta, rankrem


def _sc_topk_body(mined_hbm, npos_hbm, out_hbm, mined_v, npos_v, hist_v, outbuf_v):
    wid = lax.axis_index("s") * 2 + lax.axis_index("c")
    pltpu.sync_copy(mined_hbm.at[wid], mined_v)
    pltpu.sync_copy(npos_hbm, npos_v)

    lane = jnp.mod(wid, 16)
    half = wid // 16
    va = npos_v[pl.ds(0, 16)]
    vb = npos_v[pl.ds(16, 16)]
    hsel = jnp.full((16,), half, _i32) == 0
    nv16 = jnp.where(hsel, va, vb)
    npf = _scsum(jnp.where(lax.broadcasted_iota(_i32, (16,), 0) == lane,
                           nv16, jnp.zeros((16,), _f32)))
    k = jnp.minimum(_NEGPOS_RATIO * npf.astype(_i32), _P - 1)

    def zbody(i, c):
        hist_v[pl.ds(i * 16, 16)] = jnp.zeros((16,), _i32)
        return c

    lax.fori_loop(0, _HTOT // 16, zbody, 0)

    ones = jnp.ones((16,), _i32)

    def p1(i, c):
        v = mined_v[pl.ds(i * 16, 16)]
        b = lax.shift_right_logical(plsc.bitcast(v, _i32), 20)
        plsc.addupdate_scatter(hist_v, [b + _BASE1], ones)
        return c

    lax.fori_loop(0, _NV, p1, 0)
    b1, r2 = _locate(hist_v, _BASE1, 2048, k)

    def p2(i, c):
        v = mined_v[pl.ds(i * 16, 16)]
        bits = plsc.bitcast(v, _i32)
        pre = lax.shift_right_logical(bits, 20)
        b = jnp.bitwise_and(lax.shift_right_logical(bits, 9), 0x7FF)
        plsc.addupdate_scatter(hist_v, [b + _BASE2], ones, mask=pre == b1)
        return c

    lax.fori_loop(0, _NV, p2, 0)
    b2, r3 = _locate(hist_v, _BASE2, 2048, r2)

    pre22 = jnp.bitwise_or(lax.shift_left(b1, 11), b2)

    def p3(i, c):
        v = mined_v[pl.ds(i * 16, 16)]
        bits = plsc.bitcast(v, _i32)
        pre = lax.shift_right_logical(bits, 9)
        b = jnp.bitwise_and(bits, 0x1FF)
        plsc.addupdate_scatter(hist_v, [b + _BASE3], ones, mask=pre == pre22)
        return c

    lax.fori_loop(0, _NV, p3, 0)
    b3, _r4 = _locate(hist_v, _BASE3, 512, r3)

    tau_bits = jnp.bitwise_or(lax.shift_left(b1, 20),
                              jnp.bitwise_or(lax.shift_left(b2, 9), b3))

    def p4(i, carry):
        cs, ss = carry
        v = mined_v[pl.ds(i * 16, 16)]
        bits = plsc.bitcast(v, _i32)
        m = bits > tau_bits
        cs = cs + jnp.where(m, ones, jnp.zeros((16,), _i32))
        ss = ss + jnp.where(m, v, jnp.zeros((16,), _f32))
        return cs, ss

    cs, ss = lax.fori_loop(0, _NV, p4,
                           (jnp.zeros((16,), _i32), jnp.zeros((16,), _f32)))
    cgt = _scsum(cs)
    sgt = _scsum(ss)
    tau_v = plsc.bitcast(jnp.full((16,), tau_bits, _i32), _f32)
    res = jnp.full((16,), sgt, _f32) + \
        jnp.full((16,), (k - cgt).astype(_f32), _f32) * tau_v
    outbuf_v[...] = res
    pltpu.sync_copy(outbuf_v, out_hbm.at[wid])


def _sc_topk(mined, npos):
    mesh = plsc.VectorSubcoreMesh(core_axis_name="c", subcore_axis_name="s",
                                  num_cores=2, num_subcores=16)
    return pl.kernel(
        _sc_topk_body,
        out_type=jax.ShapeDtypeStruct((_B, 16), _f32),
        mesh=mesh,
        compiler_params=pltpu.CompilerParams(needs_layout_passes=False),
        scratch_types=[
            pltpu.VMEM((_PPAD,), _f32),
            pltpu.VMEM((_B,), _f32),
            pltpu.VMEM((_HTOT,), _i32),
            pltpu.VMEM((16,), _f32),
        ],
    )(mined, npos)


def _stage3_body(ll_ref, npos_ref, spos_ref, sneg_ref, outl_ref, outc_ref):
    n = jnp.sum(npos_ref[...])
    sneg = sneg_ref[...][:, 0:1]
    outl_ref[...] = (jnp.sum(ll_ref[...]) / n).reshape(1, 1)
    outc_ref[...] = ((jnp.sum(spos_ref[...]) + jnp.sum(sneg)) / n).reshape(1, 1)


def kernel(loc_data, conf_data, priors, targets):
    pad = _PPAD - _P
    pad_pr = jnp.tile(jnp.array([[5.0, 5.0, 0.001, 0.001]], jnp.float32), (pad, 1))
    priorsT = jnp.concatenate([priors, pad_pr], axis=0).T
    locT = jnp.pad(loc_data, ((0, 0), (0, pad), (0, 0))).transpose(0, 2, 1)
    confT = jnp.pad(conf_data, ((0, 0), (0, pad), (0, 0))).transpose(0, 2, 1)

    mined, ll, npos, spos = pl.pallas_call(
        _stage1_body,
        grid=(_B,),
        in_specs=[
            pl.BlockSpec((1, _NOBJ, 5), lambda b: (b, 0, 0)),
            pl.BlockSpec((1, 5, _NOBJ), lambda b: (b, 0, 0)),
            pl.BlockSpec((4, _PPAD), lambda b: (0, 0)),
            pl.BlockSpec((1, 4, _PPAD), lambda b: (b, 0, 0)),
            pl.BlockSpec((1, _NUM_CLASSES, _PPAD), lambda b: (b, 0, 0)),
        ],
        out_specs=[
            pl.BlockSpec((1, 1, _PPAD), lambda b: (b, 0, 0)),
            pl.BlockSpec((1, 1, 1), lambda b: (b, 0, 0)),
            pl.BlockSpec((1, 1, 1), lambda b: (b, 0, 0)),
            pl.BlockSpec((1, 1, 1), lambda b: (b, 0, 0)),
        ],
        out_shape=[
            jax.ShapeDtypeStruct((_B, 1, _PPAD), jnp.float32),
            jax.ShapeDtypeStruct((_B, 1, 1), jnp.float32),
            jax.ShapeDtypeStruct((_B, 1, 1), jnp.float32),
            jax.ShapeDtypeStruct((_B, 1, 1), jnp.float32),
        ],
        scratch_shapes=[
            pltpu.VMEM((1, _PPAD), jnp.float32),
            pltpu.VMEM((1, _PPAD), jnp.float32),
        ],
    )(targets, targets.transpose(0, 2, 1), priorsT, locT, confT)

    sneg = _sc_topk(mined.reshape(_B, _PPAD), npos.reshape(_B))

    outl, outc = pl.pallas_call(
        _stage3_body,
        out_shape=[
            jax.ShapeDtypeStruct((1, 1), jnp.float32),
            jax.ShapeDtypeStruct((1, 1), jnp.float32),
        ],
    )(ll, npos, spos, sneg)
    return outl[0, 0], outc[0, 0]


# speedup vs baseline: 106.2942x; 1.0576x over previous
import jax
import jax.numpy as jnp
from jax import lax
from jax.experimental import pallas as pl
from jax.experimental.pallas import tpu as pltpu
from jax.experimental.pallas import tpu_sc as plsc

_NUM_CLASSES = 21
_THRESHOLD = 0.5
_NEGPOS_RATIO = 3
_VAR0, _VAR1 = 0.1, 0.2
_B, _P, _NOBJ = 32, 24532, 32
_PPAD = 24576
_PB = 2048
_NCH = _PPAD // _PB

_i32 = jnp.int32
_f32 = jnp.float32


def _stage1_body(tgt_ref, tgtT_ref, pr_ref, locT_ref,
                 conft_ref, ll_ref, npos_ref,
                 bto_s, bti_s):
    f32 = jnp.float32
    t = tgt_ref[0]
    t5 = tgtT_ref[0]
    tx1, ty1 = t[:, 0:1], t[:, 1:2]
    tx2, ty2 = t[:, 2:3], t[:, 3:4]
    lab = t[:, 4:5]
    area_t = (tx2 - tx1) * (ty2 - ty1)
    jcol = jax.lax.broadcasted_iota(jnp.int32, (_NOBJ, 1), 0).astype(f32)

    runmax = jnp.full((_NOBJ, 1), -jnp.inf, f32)
    runidx = jnp.zeros((_NOBJ, 1), f32)

    for c in range(_NCH):
        sl = pl.ds(c * _PB, _PB)
        cx, cy = pr_ref[0:1, sl], pr_ref[1:2, sl]
        w, h = pr_ref[2:3, sl], pr_ref[3:4, sl]
        px1, py1 = cx - w * 0.5, cy - h * 0.5
        px2, py2 = w + w * 0.5, h + h * 0.5
        area_p = (px2 - px1) * (py2 - py1)
        ix = jnp.maximum(jnp.minimum(tx2, px2) - jnp.maximum(tx1, px1), 0.0)
        iy = jnp.maximum(jnp.minimum(ty2, py2) - jnp.maximum(ty1, py1), 0.0)
        inter = ix * iy
        ov = inter / (area_t + area_p - inter)
        bto_c = jnp.max(ov, axis=0, keepdims=True)
        bti_c = jnp.min(jnp.where(ov == bto_c, jcol, 99.0), axis=0, keepdims=True)
        bto_s[:, sl] = bto_c
        bti_s[:, sl] = bti_c
        gidx = jax.lax.broadcasted_iota(jnp.int32, (1, _PB), 1).astype(f32) + float(c * _PB)
        rm_c = jnp.max(ov, axis=1, keepdims=True)
        ri_c = jnp.min(jnp.where(ov == rm_c, gidx, 1e9), axis=1, keepdims=True)
        upd = rm_c > runmax
        runmax = jnp.where(upd, rm_c, runmax)
        runidx = jnp.where(upd, ri_c, runidx)

    bpi = runidx

    ll = jnp.zeros((), f32)
    np_ = jnp.zeros((), f32)

    for c in range(_NCH):
        sl = pl.ds(c * _PB, _PB)
        gidx = jax.lax.broadcasted_iota(jnp.int32, (1, _PB), 1).astype(f32) + float(c * _PB)
        bto = bto_s[:, sl]
        bti = bti_s[:, sl]
        hit = gidx == bpi
        forcej = jnp.max(jnp.where(hit, jcol, -1.0), axis=0, keepdims=True)
        forced = forcej >= 0.0
        bto = jnp.where(forced, 2.0, bto)
        bti = jnp.where(forced, forcej, bti)
        selm = (bti == jcol).astype(f32)
        coords = jax.lax.dot_general(t5, selm, (((1,), (0,)), ((), ())),
                                     preferred_element_type=f32)
        mx1, my1 = coords[0:1, :], coords[1:2, :]
        mx2, my2 = coords[2:3, :], coords[3:4, :]
        labm = coords[4:5, :]
        conf = jnp.where(bto < _THRESHOLD, 0.0, labm + 1.0)
        pos = conf > 0.0
        posf = pos.astype(f32)
        cx, cy = pr_ref[0:1, sl], pr_ref[1:2, sl]
        w, h = pr_ref[2:3, sl], pr_ref[3:4, sl]
        g0 = (mx1 + mx2 * 0.5 - cx) / (_VAR0 * w)
        g1 = (my1 + my2 * 0.5 - cy) / (_VAR0 * h)
        g2 = jnp.log((mx2 - mx1) / w) / _VAR1
        g3 = jnp.log((my2 - my1) / h) / _VAR1
        loct = jnp.concatenate([g0, g1, g2, g3], axis=0)
        locd = locT_ref[0, :, sl]
        ad = jnp.abs(locd - loct)
        sl1 = jnp.where(ad < 1.0, 0.5 * ad * ad, ad - 0.5)
        ll = ll + jnp.sum(sl1 * posf)
        np_ = np_ + jnp.sum(posf)
        conft_ref[0, :, sl] = conf

    ll_ref[...] = ll.reshape(1, 1, 1)
    npos_ref[...] = np_.reshape(1, 1, 1)


def _stage1b_body(confT_ref, conft_ref, mined_ref, spos_ref):
    f32 = jnp.float32
    sp = jnp.zeros((), f32)
    clsio = jax.lax.broadcasted_iota(jnp.int32, (_NUM_CLASSES, 1), 0).astype(f32)
    for c in range(_NCH):
        sl = pl.ds(c * _PB, _PB)
        gidx = jax.lax.broadcasted_iota(jnp.int32, (1, _PB), 1).astype(f32) + float(c * _PB)
        conf = conft_ref[0, :, sl]
        pos = conf > 0.0
        x = confT_ref[0, :, sl]
        m = jnp.max(x, axis=0, keepdims=True)
        s = jnp.sum(jnp.exp(x - m), axis=0, keepdims=True)
        lse = jnp.log(s) + m
        tgt = jnp.sum(jnp.where(clsio == conf, x, 0.0), axis=0, keepdims=True)
        ce = lse - tgt
        mined = jnp.where(pos, 0.0, ce)
        mined = jnp.where(gidx < float(_P), mined, 0.0)
        mined_ref[0, :, sl] = mined
        sp = sp + jnp.sum(jnp.where(pos, ce, 0.0))

    spos_ref[...] = sp.reshape(1, 1, 1)


_NV = _PPAD // 16
_BASE1, _BASE2, _BASE3 = 0, 2048, 4096
_HTOT = 4608


def _scsum(v):
    return lax.reduce_sum(v, (0,))


def _lane_extract(vec, lane):
    io = lax.broadcasted_iota(_i32, (16,), 0)
    return _scsum(jnp.where(io == lane, vec, jnp.zeros_like(vec)))


def _locate(hist_v, base, nbuckets, rank):
    nv = nbuckets // 16

    def body(i, carry):
        found, beta, rankrem, acc = carry
        vi = nv - 1 - i
        h = hist_v[pl.ds(base + vi * 16, 16)]
        s = lax.rev(lax.cumsum(lax.rev(h, (0,)), axis=0), (0,))
        suff = s + acc
        m = suff >= rank
        t = _scsum(jnp.where(m, jnp.ones((16,), _i32), jnp.zeros((16,), _i32)))
        hit = jnp.logical_and(found == 0, t > 0)
        lane = t - 1
        sl = _lane_extract(s, lane)
        hl = _lane_extract(h, lane)
        beta_new = vi * 16 + lane
        rr_new = rank - (acc + sl - hl)
        found = jnp.where(hit, 1, found)
        beta = jnp.where(hit, beta_new, beta)
        rankrem = jnp.where(hit, rr_new, rankrem)
        acc = acc + _scsum(h)
        return found, beta, rankrem, acc

    z = jnp.zeros((), _i32)
    found, beta, rankrem, acc = lax.fori_loop(0, nv, body, (z, z, z, z))
    return beta, rankrem


def _sc_topk_body(mined_hbm, npos_hbm, out_hbm, mined_v, npos_v, hist_v, outbuf_v):
    wid = lax.axis_index("s") * 2 + lax.axis_index("c")
    pltpu.sync_copy(mined_hbm.at[wid], mined_v)
    pltpu.sync_copy(npos_hbm, npos_v)

    lane = jnp.mod(wid, 16)
    half = wid // 16
    va = npos_v[pl.ds(0, 16)]
    vb = npos_v[pl.ds(16, 16)]
    hsel = jnp.full((16,), half, _i32) == 0
    nv16 = jnp.where(hsel, va, vb)
    npf = _scsum(jnp.where(lax.broadcasted_iota(_i32, (16,), 0) == lane,
                           nv16, jnp.zeros((16,), _f32)))
    k = jnp.minimum(_NEGPOS_RATIO * npf.astype(_i32), _P - 1)

    def zbody(i, c):
        hist_v[pl.ds(i * 16, 16)] = jnp.zeros((16,), _i32)
        return c

    lax.fori_loop(0, _HTOT // 16, zbody, 0)

    ones = jnp.ones((16,), _i32)

    def p1(i, c):
        v = mined_v[pl.ds(i * 16, 16)]
        b = lax.shift_right_logical(plsc.bitcast(v, _i32), 20)
        plsc.addupdate_scatter(hist_v, [b + _BASE1], ones)
        return c

    lax.fori_loop(0, _NV, p1, 0)
    b1, r2 = _locate(hist_v, _BASE1, 2048, k)

    def p2(i, c):
        v = mined_v[pl.ds(i * 16, 16)]
        bits = plsc.bitcast(v, _i32)
        pre = lax.shift_right_logical(bits, 20)
        b = jnp.bitwise_and(lax.shift_right_logical(bits, 9), 0x7FF)
        plsc.addupdate_scatter(hist_v, [b + _BASE2], ones, mask=pre == b1)
        return c

    lax.fori_loop(0, _NV, p2, 0)
    b2, r3 = _locate(hist_v, _BASE2, 2048, r2)

    pre22 = jnp.bitwise_or(lax.shift_left(b1, 11), b2)

    def p3(i, c):
        v = mined_v[pl.ds(i * 16, 16)]
        bits = plsc.bitcast(v, _i32)
        pre = lax.shift_right_logical(bits, 9)
        b = jnp.bitwise_and(bits, 0x1FF)
        plsc.addupdate_scatter(hist_v, [b + _BASE3], ones, mask=pre == pre22)
        return c

    lax.fori_loop(0, _NV, p3, 0)
    b3, _r4 = _locate(hist_v, _BASE3, 512, r3)

    tau_bits = jnp.bitwise_or(lax.shift_left(b1, 20),
                              jnp.bitwise_or(lax.shift_left(b2, 9), b3))

    def p4(i, carry):
        cs, ss = carry
        v = mined_v[pl.ds(i * 16, 16)]
        bits = plsc.bitcast(v, _i32)
        m = bits > tau_bits
        cs = cs + jnp.where(m, ones, jnp.zeros((16,), _i32))
        ss = ss + jnp.where(m, v, jnp.zeros((16,), _f32))
        return cs, ss

    cs, ss = lax.fori_loop(0, _NV, p4,
                           (jnp.zeros((16,), _i32), jnp.zeros((16,), _f32)))
    cgt = _scsum(cs)
    sgt = _scsum(ss)
    tau_v = plsc.bitcast(jnp.full((16,), tau_bits, _i32), _f32)
    res = jnp.full((16,), sgt, _f32) + \
        jnp.full((16,), (k - cgt).astype(_f32), _f32) * tau_v
    outbuf_v[...] = res
    pltpu.sync_copy(outbuf_v, out_hbm.at[wid])


def _sc_topk(mined, npos):
    mesh = plsc.VectorSubcoreMesh(core_axis_name="c", subcore_axis_name="s",
                                  num_cores=2, num_subcores=16)
    return pl.kernel(
        _sc_topk_body,
        out_type=jax.ShapeDtypeStruct((_B, 16), _f32),
        mesh=mesh,
        compiler_params=pltpu.CompilerParams(needs_layout_passes=False),
        scratch_types=[
            pltpu.VMEM((_PPAD,), _f32),
            pltpu.VMEM((_B,), _f32),
            pltpu.VMEM((_HTOT,), _i32),
            pltpu.VMEM((16,), _f32),
        ],
    )(mined, npos)


def _stage3_body(ll_ref, npos_ref, spos_ref, sneg_ref, outl_ref, outc_ref):
    n = jnp.sum(npos_ref[...])
    sneg = sneg_ref[...][:, 0:1]
    outl_ref[...] = (jnp.sum(ll_ref[...]) / n).reshape(1, 1)
    outc_ref[...] = ((jnp.sum(spos_ref[...]) + jnp.sum(sneg)) / n).reshape(1, 1)


def kernel(loc_data, conf_data, priors, targets):
    pad = _PPAD - _P
    pad_pr = jnp.tile(jnp.array([[5.0, 5.0, 0.001, 0.001]], jnp.float32), (pad, 1))
    priorsT = jnp.concatenate([priors, pad_pr], axis=0).T
    locT = jnp.pad(loc_data, ((0, 0), (0, pad), (0, 0))).transpose(0, 2, 1)
    confT = jnp.pad(conf_data, ((0, 0), (0, pad), (0, 0))).transpose(0, 2, 1)

    conft, ll, npos = pl.pallas_call(
        _stage1_body,
        grid=(_B,),
        in_specs=[
            pl.BlockSpec((1, _NOBJ, 5), lambda b: (b, 0, 0)),
            pl.BlockSpec((1, 5, _NOBJ), lambda b: (b, 0, 0)),
            pl.BlockSpec((4, _PPAD), lambda b: (0, 0)),
            pl.BlockSpec((1, 4, _PPAD), lambda b: (b, 0, 0)),
        ],
        out_specs=[
            pl.BlockSpec((1, 1, _PPAD), lambda b: (b, 0, 0)),
            pl.BlockSpec((1, 1, 1), lambda b: (b, 0, 0)),
            pl.BlockSpec((1, 1, 1), lambda b: (b, 0, 0)),
        ],
        out_shape=[
            jax.ShapeDtypeStruct((_B, 1, _PPAD), jnp.float32),
            jax.ShapeDtypeStruct((_B, 1, 1), jnp.float32),
            jax.ShapeDtypeStruct((_B, 1, 1), jnp.float32),
        ],
        scratch_shapes=[
            pltpu.VMEM((1, _PPAD), jnp.float32),
            pltpu.VMEM((1, _PPAD), jnp.float32),
        ],
    )(targets, targets.transpose(0, 2, 1), priorsT, locT)

    mined, spos = pl.pallas_call(
        _stage1b_body,
        grid=(_B,),
        in_specs=[
            pl.BlockSpec((1, _NUM_CLASSES, _PPAD), lambda b: (b, 0, 0)),
            pl.BlockSpec((1, 1, _PPAD), lambda b: (b, 0, 0)),
        ],
        out_specs=[
            pl.BlockSpec((1, 1, _PPAD), lambda b: (b, 0, 0)),
            pl.BlockSpec((1, 1, 1), lambda b: (b, 0, 0)),
        ],
        out_shape=[
            jax.ShapeDtypeStruct((_B, 1, _PPAD), jnp.float32),
            jax.ShapeDtypeStruct((_B, 1, 1), jnp.float32),
        ],
    )(confT, conft)

    sneg = _sc_topk(mined.reshape(_B, _PPAD), npos.reshape(_B))

    outl, outc = pl.pallas_call(
        _stage3_body,
        out_shape=[
            jax.ShapeDtypeStruct((1, 1), jnp.float32),
            jax.ShapeDtypeStruct((1, 1), jnp.float32),
        ],
    )(ll, npos, spos, sneg)
    return outl[0, 0], outc[0, 0]
